# Initial kernel scaffold; baseline (speedup 1.0000x reference)
import jax
import jax.numpy as jnp
from jax.experimental import pallas as pl


def kernel(x, edge_index, edge_score, batch, conv1_w1, conv1_b1, conv1_w2, conv1_w3, conv1_b3, conv2_w1, conv2_b1, conv2_w2, conv2_w3, conv2_b3, conv3_w1, conv3_b1, conv3_w2, conv3_w3, conv3_b3, conv4_w1, conv4_b1, conv4_w2, conv4_w3, conv4_b3, mlp_w1, mlp_b1, mlp_w2, mlp_b2):
    raise NotImplementedError("write your pallas kernel here")


# speedup vs baseline: 3.1717x; 3.1717x over previous
import jax
import jax.numpy as jnp
from jax import lax
from jax.experimental import pallas as pl
from jax.experimental.pallas import tpu as pltpu
from jax.experimental.pallas import tpu_sc as plsc

_N = 50000
_E = 800000
_CH = 64
_G = 128
_NC = 2
_NS = 16
_NPAD = 50176
_EPAD = 802816
_K = 128
_ZR = 784
_RPT = _NPAD // _NS
_GPAD = 136
_CK = 448

_mesh = lambda: plsc.VectorSubcoreMesh(core_axis_name="c", subcore_axis_name="s")


def _zero_zbuf(zbuf, width):
    nv = width // 16

    def zb(k, _):
        for j in range(nv):
            zbuf[k, pl.ds(16 * j, 16)] = jnp.zeros((16,), jnp.float32)
        return 0

    lax.fori_loop(0, _ZR, zb, 0)


def _zero_acc(zbuf, acc, s):
    def cz(k, _):
        pltpu.sync_copy(zbuf, acc.at[pl.ds(s * _RPT + k * _ZR, _ZR)])
        return 0

    lax.fori_loop(0, _RPT // _ZR, cz, 0)


def _readout(acc, out, cc, s):
    def ro(k, _):
        r0 = s * _RPT + k * _ZR
        pltpu.sync_copy(acc.at[pl.ds(r0, _ZR)], out.at[cc, pl.ds(r0, _ZR)])
        return 0

    lax.fori_loop(0, _RPT // _ZR, ro, 0)


def _pass_a(xp, esf, srcp, dstp):
    def body(xp_h, esf_h, src_h, dst_h, so, sbuf, dbuf, esb, ewb,
             xrows, msgb, accA, zbuf, sem):
        c = lax.axis_index("c")
        s = lax.axis_index("s")
        _zero_zbuf(zbuf, 16)
        _zero_acc(zbuf, accA, s)

        def zs(e, _):
            msgb[e, :] = jnp.zeros((16,), jnp.float32)
            return 0

        lax.fori_loop(0, _K, zs, 0)
        plsc.subcore_barrier()

        ept = _EPAD // (_NC * _NS)
        tile_base = (c * _NS + s) * ept
        iot = lax.iota(jnp.int32, 16)

        def chunk(kk, _):
            e0 = tile_base + kk * _K
            pltpu.sync_copy(src_h.at[pl.ds(e0, _K)], sbuf)
            pltpu.sync_copy(dst_h.at[pl.ds(e0, _K)], dbuf)
            pltpu.sync_copy(esf_h.at[pl.ds(e0 * 4, _K * 4)], esb)
            gth = pltpu.async_copy(xp_h.at[sbuf], xrows, sem)

            def sg(r, _):
                v = esb[pl.ds(r * 16, 16)]
                sig = 1.0 / (1.0 + jnp.exp(-v))
                rows_i = 4 * r + iot // 4
                cols_i = 4 + iot % 4
                plsc.store_scatter(msgb, [rows_i, cols_i], sig)
                plsc.store_scatter(ewb, [rows_i], sig, mask=(iot % 4) == 0)
                return 0

            lax.fori_loop(0, (_K * 4) // 16, sg, 0)
            gth.wait()

            def pe(e, _):
                ei = jnp.full((16,), e, jnp.int32)
                w = plsc.load_gather(ewb, [ei])
                prod = xrows[e, :] * w
                plsc.store_scatter(msgb, [ei, iot], prod, mask=iot < 4)
                return 0

            lax.fori_loop(0, _K, pe, 0)

            pltpu.async_copy(msgb, accA.at[dbuf], sem, add=True).wait()
            return 0

        lax.fori_loop(0, ept // _K, chunk, 0)
        plsc.subcore_barrier()

        for cc in range(_NC):
            @pl.when(c == cc)
            def _():
                _readout(accA, so, cc, s)

    f = pl.kernel(
        body,
        out_type=jax.ShapeDtypeStruct((_NC, _NPAD, 16), jnp.float32),
        mesh=_mesh(),
        compiler_params=pltpu.CompilerParams(
            needs_layout_passes=False, use_tc_tiling_on_sc=False),
        scratch_types=[
            pltpu.VMEM((_K,), jnp.int32),
            pltpu.VMEM((_K,), jnp.int32),
            pltpu.VMEM((_K * 4,), jnp.float32),
            pltpu.VMEM((_K,), jnp.float32),
            pltpu.VMEM((_K, 16), jnp.float32),
            pltpu.VMEM((_K, 16), jnp.float32),
            pltpu.VMEM_SHARED((_NPAD, 16), jnp.float32),
            pltpu.VMEM((_ZR, 16), jnp.float32),
            pltpu.SemaphoreType.DMA,
        ],
    )
    return f(xp, esf, srcp, dstp)


def _pass_b(col, hA, hB, esT, srcp, dstp):
    def body(hA_h, hB_h, esT_h, src_h, dst_h, so, sbuf, dbuf, esb, ewb,
             rows, accB, zbuf, sem):
        c = lax.axis_index("c")
        s = lax.axis_index("s")
        _zero_zbuf(zbuf, 32)
        _zero_acc(zbuf, accB, s)
        plsc.subcore_barrier()

        ept = _EPAD // _NS

        def half(h_h):
            def chunk(kk, _):
                e0 = s * ept + kk * _K
                pltpu.sync_copy(src_h.at[pl.ds(e0, _K)], sbuf)
                pltpu.sync_copy(dst_h.at[pl.ds(e0, _K)], dbuf)
                pltpu.sync_copy(esT_h.at[col, pl.ds(e0, _K)], esb)
                pltpu.async_copy(h_h.at[sbuf], rows, sem).wait()

                def sg(r, _):
                    v = esb[pl.ds(r * 16, 16)]
                    ewb[pl.ds(r * 16, 16)] = 1.0 / (1.0 + jnp.exp(-v))
                    return 0

                lax.fori_loop(0, _K // 16, sg, 0)

                def pe(e, _):
                    w = plsc.load_gather(ewb, [jnp.full((16,), e, jnp.int32)])
                    r0 = rows[e, pl.ds(0, 16)]
                    rows[e, pl.ds(0, 16)] = r0 * w
                    r1 = rows[e, pl.ds(16, 16)]
                    rows[e, pl.ds(16, 16)] = r1 * w
                    return 0

                lax.fori_loop(0, _K, pe, 0)

                pltpu.async_copy(rows, accB.at[dbuf], sem, add=True).wait()
                return 0

            lax.fori_loop(0, ept // _K, chunk, 0)

        @pl.when(c == 0)
        def _():
            half(hA_h)

        @pl.when(c == 1)
        def _():
            half(hB_h)

        plsc.subcore_barrier()
        for cc in range(_NC):
            @pl.when(c == cc)
            def _():
                _readout(accB, so, cc, s)

    f = pl.kernel(
        body,
        out_type=jax.ShapeDtypeStruct((_NC, _NPAD, 32), jnp.float32),
        mesh=_mesh(),
        compiler_params=pltpu.CompilerParams(
            needs_layout_passes=False, use_tc_tiling_on_sc=False),
        scratch_types=[
            pltpu.VMEM((_K,), jnp.int32),
            pltpu.VMEM((_K,), jnp.int32),
            pltpu.VMEM((_K,), jnp.float32),
            pltpu.VMEM((_K,), jnp.float32),
            pltpu.VMEM((_K, 32), jnp.float32),
            pltpu.VMEM_SHARED((_NPAD, 32), jnp.float32),
            pltpu.VMEM((_ZR, 32), jnp.float32),
            pltpu.SemaphoreType.DMA,
        ],
    )
    return f(hA, hB, esT, srcp, dstp)


def _pass_c(hA, hB, batchp):
    def body(hA_h, hB_h, b_h, pool, cnto, hbuf, bbuf, ones, accP, accC, sem):
        c = lax.axis_index("c")
        s = lax.axis_index("s")

        def zh(k, _):
            hbuf[k, pl.ds(0, 16)] = jnp.zeros((16,), jnp.float32)
            hbuf[k, pl.ds(16, 16)] = jnp.zeros((16,), jnp.float32)
            ones[k, :] = jnp.zeros((16,), jnp.float32)
            return 0

        lax.fori_loop(0, _GPAD, zh, 0)

        @pl.when(s == 0)
        def _():
            pltpu.sync_copy(hbuf.at[pl.ds(0, _GPAD)], accP)
            pltpu.sync_copy(ones.at[pl.ds(0, _GPAD)], accC)

        plsc.subcore_barrier()

        def fo(k, _):
            ones[k, :] = jnp.ones((16,), jnp.float32)
            return 0

        lax.fori_loop(0, _CK, fo, 0)

        def half(cc, h_h):
            def chunk(kk, _):
                r0 = s * _RPT + kk * _CK
                pltpu.sync_copy(h_h.at[pl.ds(r0, _CK)], hbuf)
                pltpu.sync_copy(b_h.at[pl.ds(r0, _CK)], bbuf)
                pltpu.async_copy(hbuf, accP.at[bbuf], sem, add=True).wait()
                if cc == 0:
                    pltpu.async_copy(ones, accC.at[bbuf], sem, add=True).wait()
                return 0

            lax.fori_loop(0, _RPT // _CK, chunk, 0)

        @pl.when(c == 0)
        def _():
            half(0, hA_h)

        @pl.when(c == 1)
        def _():
            half(1, hB_h)

        plsc.subcore_barrier()
        for cc in range(_NC):
            @pl.when((c == cc) & (s == 0))
            def _():
                pltpu.sync_copy(accP, pool.at[cc])

        @pl.when((c == 0) & (s == 0))
        def _():
            pltpu.sync_copy(accC, cnto)

    f = pl.kernel(
        body,
        out_type=[
            jax.ShapeDtypeStruct((_NC, _GPAD, 32), jnp.float32),
            jax.ShapeDtypeStruct((_GPAD, 16), jnp.float32),
        ],
        mesh=_mesh(),
        compiler_params=pltpu.CompilerParams(
            needs_layout_passes=False, use_tc_tiling_on_sc=False),
        scratch_types=[
            pltpu.VMEM((_CK, 32), jnp.float32),
            pltpu.VMEM((_CK,), jnp.int32),
            pltpu.VMEM((_CK, 16), jnp.float32),
            pltpu.VMEM_SHARED((_GPAD, 32), jnp.float32),
            pltpu.VMEM_SHARED((_GPAD, 16), jnp.float32),
            pltpu.SemaphoreType.DMA,
        ],
    )
    return f(hA, hB, batchp)


_RT = 1568


def _tc1(xp, s1d, w1, b1, w2, w3, b3):
    def body(x_r, s_r, w1r, b1r, w2r, w3r, b3r, oa, ob, od):
        C = s_r[0] + s_r[1]
        S = C[:, 0:4]
        d0 = C[:, 4:5]
        xb = x_r[:, 0:4]
        h = (jnp.dot(S, w1r[...].T, preferred_element_type=jnp.float32)
             + d0 * b1r[...][None, :]
             - d0 * jnp.dot(xb, w2r[...].T, preferred_element_type=jnp.float32)
             + jnp.dot(xb, w3r[...].T, preferred_element_type=jnp.float32)
             + b3r[...][None, :])
        h = jnp.maximum(h, 0.0)
        oa[...] = h[:, 0:32]
        ob[...] = h[:, 32:64]
        od[...] = C

    return pl.pallas_call(
        body,
        grid=(_NPAD // _RT,),
        in_specs=[
            pl.BlockSpec((_RT, 16), lambda i: (i, 0)),
            pl.BlockSpec((_NC, _RT, 16), lambda i: (0, i, 0)),
            pl.BlockSpec((_CH, 4), lambda i: (0, 0)),
            pl.BlockSpec((_CH,), lambda i: (0,)),
            pl.BlockSpec((_CH, 4), lambda i: (0, 0)),
            pl.BlockSpec((_CH, 4), lambda i: (0, 0)),
            pl.BlockSpec((_CH,), lambda i: (0,)),
        ],
        out_specs=[
            pl.BlockSpec((_RT, 32), lambda i: (i, 0)),
            pl.BlockSpec((_RT, 32), lambda i: (i, 0)),
            pl.BlockSpec((_RT, 16), lambda i: (i, 0)),
        ],
        out_shape=[
            jax.ShapeDtypeStruct((_NPAD, 32), jnp.float32),
            jax.ShapeDtypeStruct((_NPAD, 32), jnp.float32),
            jax.ShapeDtypeStruct((_NPAD, 16), jnp.float32),
        ],
    )(xp, s1d, w1, b1, w2, w3, b3)


def _tc_mid(col, hA, hB, sp, dsum, w1, b1, w2, w3, b3):
    def body(ha_r, hb_r, s_r, d_r, w1r, b1r, w2r, w3r, b3r, oa, ob):
        h = jnp.concatenate([ha_r[...], hb_r[...]], axis=1)
        S = jnp.concatenate([s_r[0], s_r[1]], axis=1)
        d = d_r[:, 4 + col:5 + col]
        hn = (jnp.dot(S, w1r[...].T, preferred_element_type=jnp.float32)
              + d * b1r[...][None, :]
              - d * jnp.dot(h, w2r[...].T, preferred_element_type=jnp.float32)
              + jnp.dot(h, w3r[...].T, preferred_element_type=jnp.float32)
              + b3r[...][None, :])
        hn = jnp.maximum(hn, 0.0)
        oa[...] = hn[:, 0:32]
        ob[...] = hn[:, 32:64]

    return pl.pallas_call(
        body,
        grid=(_NPAD // _RT,),
        in_specs=[
            pl.BlockSpec((_RT, 32), lambda i: (i, 0)),
            pl.BlockSpec((_RT, 32), lambda i: (i, 0)),
            pl.BlockSpec((_NC, _RT, 32), lambda i: (0, i, 0)),
            pl.BlockSpec((_RT, 16), lambda i: (i, 0)),
            pl.BlockSpec((_CH, _CH), lambda i: (0, 0)),
            pl.BlockSpec((_CH,), lambda i: (0,)),
            pl.BlockSpec((_CH, _CH), lambda i: (0, 0)),
            pl.BlockSpec((_CH, _CH), lambda i: (0, 0)),
            pl.BlockSpec((_CH,), lambda i: (0,)),
        ],
        out_specs=[
            pl.BlockSpec((_RT, 32), lambda i: (i, 0)),
            pl.BlockSpec((_RT, 32), lambda i: (i, 0)),
        ],
        out_shape=[
            jax.ShapeDtypeStruct((_NPAD, 32), jnp.float32),
            jax.ShapeDtypeStruct((_NPAD, 32), jnp.float32),
        ],
    )(hA, hB, sp, dsum, w1, b1, w2, w3, b3)


def _tc_final(pool, cnt, w1, b1, w2, b2):
    def body(p_r, c_r, w1r, b1r, w2r, b2r, ogf, opred):
        sums = jnp.concatenate([p_r[0, 0:_G, :], p_r[1, 0:_G, :]], axis=1)
        cv = c_r[0:_G, 0:1]
        gf = sums / jnp.maximum(cv, 1.0)
        hid = jnp.maximum(
            jnp.dot(gf, w1r[...].T, preferred_element_type=jnp.float32)
            + b1r[...][None, :], 0.0)
        pred = (jnp.dot(hid, w2r[...].T, preferred_element_type=jnp.float32)
                + b2r[...][None, :])
        ogf[...] = gf
        opred[...] = pred

    return pl.pallas_call(
        body,
        out_shape=[
            jax.ShapeDtypeStruct((_G, _CH), jnp.float32),
            jax.ShapeDtypeStruct((_G, 2), jnp.float32),
        ],
    )(pool, cnt, w1, b1, w2, b2)


def kernel(x, edge_index, edge_score, batch,
           conv1_w1, conv1_b1, conv1_w2, conv1_w3, conv1_b3,
           conv2_w1, conv2_b1, conv2_w2, conv2_w3, conv2_b3,
           conv3_w1, conv3_b1, conv3_w2, conv3_w3, conv3_b3,
           conv4_w1, conv4_b1, conv4_w2, conv4_w3, conv4_b3,
           mlp_w1, mlp_b1, mlp_w2, mlp_b2):
    src = edge_index[0]
    dst = edge_index[1]
    pad_e = _EPAD - _E
    srcp = jnp.concatenate([src, jnp.zeros((pad_e,), jnp.int32)])
    dstp = jnp.concatenate([dst, jnp.full((pad_e,), _NPAD - 1, jnp.int32)])
    esp = jnp.concatenate(
        [edge_score, jnp.zeros((pad_e, 4), jnp.float32)], axis=0)
    esf = esp.reshape(-1)
    esT = esp.T
    xp = jnp.zeros((_NPAD, 16), jnp.float32).at[:_N, 0:4].set(x)
    batchp = jnp.concatenate([batch, jnp.full((_NPAD - _N,), _G, jnp.int32)])

    s1d = _pass_a(xp, esf, srcp, dstp)
    hA, hB, dsum = _tc1(xp, s1d, conv1_w1, conv1_b1, conv1_w2,
                        conv1_w3, conv1_b3)
    sp = _pass_b(1, hA, hB, esT, srcp, dstp)
    hA, hB = _tc_mid(1, hA, hB, sp, dsum, conv2_w1, conv2_b1, conv2_w2,
                     conv2_w3, conv2_b3)
    sp = _pass_b(2, hA, hB, esT, srcp, dstp)
    hA, hB = _tc_mid(2, hA, hB, sp, dsum, conv3_w1, conv3_b1, conv3_w2,
                     conv3_w3, conv3_b3)
    sp = _pass_b(3, hA, hB, esT, srcp, dstp)
    hA, hB = _tc_mid(3, hA, hB, sp, dsum, conv4_w1, conv4_b1, conv4_w2,
                     conv4_w3, conv4_b3)
    pool, cnt = _pass_c(hA, hB, batchp)
    gf, pred = _tc_final(pool, cnt, mlp_w1, mlp_b1, mlp_w2, mlp_b2)
    return (gf, pred)


# speedup vs baseline: 4.3451x; 1.3700x over previous
import jax
import jax.numpy as jnp
from jax import lax
from jax.experimental import pallas as pl
from jax.experimental.pallas import tpu as pltpu
from jax.experimental.pallas import tpu_sc as plsc

_N = 50000
_E = 800000
_CH = 64
_G = 128
_NC = 2
_NS = 16
_NPAD = 50176
_EPAD = 802816
_K = 64
_ZR = 784
_RPT = _NPAD // _NS
_GPAD = 136
_CK = 448

_mesh = lambda: plsc.VectorSubcoreMesh(core_axis_name="c", subcore_axis_name="s")


def _zero_zbuf(zbuf, width):
    nv = width // 16

    def zb(k, _):
        for j in range(nv):
            zbuf[k, pl.ds(16 * j, 16)] = jnp.zeros((16,), jnp.float32)
        return 0

    lax.fori_loop(0, _ZR, zb, 0)


def _zero_acc(zbuf, acc, s):
    def cz(k, _):
        pltpu.sync_copy(zbuf, acc.at[pl.ds(s * _RPT + k * _ZR, _ZR)])
        return 0

    lax.fori_loop(0, _RPT // _ZR, cz, 0)


def _readout(acc, out, cc, s):
    def ro(k, _):
        r0 = s * _RPT + k * _ZR
        pltpu.sync_copy(acc.at[pl.ds(r0, _ZR)], out.at[cc, pl.ds(r0, _ZR)])
        return 0

    lax.fori_loop(0, _RPT // _ZR, ro, 0)


_KA = 448


def _pass_a(xp, esf, srcp, dstp):
    def body(xp_h, esf_h, src_h, dst_h, so,
             sbuf0, sbuf1, dbuf0, dbuf1, esb0, esb1, ewb, xrows,
             msgb0, msgb1, accA, zbuf,
             semL0, semL1, semG, semS0, semS1):
        c = lax.axis_index("c")
        s = lax.axis_index("s")
        _zero_zbuf(zbuf, 16)
        _zero_acc(zbuf, accA, s)

        sb = (sbuf0, sbuf1)
        db = (dbuf0, dbuf1)
        eb = (esb0, esb1)
        mb = (msgb0, msgb1)
        sL = (semL0, semL1)
        sS = (semS0, semS1)

        def zs(e, _):
            msgb0[e, :] = jnp.zeros((16,), jnp.float32)
            msgb1[e, :] = jnp.zeros((16,), jnp.float32)
            return 0

        lax.fori_loop(0, _KA, zs, 0)
        plsc.subcore_barrier()

        ept = _EPAD // (_NC * _NS)
        tile_base = (c * _NS + s) * ept
        nch = ept // _KA
        iot = lax.iota(jnp.int32, 16)

        def issueL(kk, b):
            e0 = tile_base + kk * _KA
            pltpu.async_copy(src_h.at[pl.ds(e0, _KA)], sb[b], sL[b])
            pltpu.async_copy(dst_h.at[pl.ds(e0, _KA)], db[b], sL[b])
            pltpu.async_copy(esf_h.at[pl.ds(e0 * 4, _KA * 4)], eb[b], sL[b])

        def waitL(b):
            pltpu.make_async_copy(src_h.at[pl.ds(0, _KA)], sb[b], sL[b]).wait()
            pltpu.make_async_copy(src_h.at[pl.ds(0, _KA)], db[b], sL[b]).wait()
            pltpu.make_async_copy(esf_h.at[pl.ds(0, _KA * 4)], eb[b], sL[b]).wait()

        def waitS(b):
            pltpu.make_async_copy(xp_h.at[pl.ds(0, _KA)], mb[b], sS[b]).wait()

        issueL(0, 0)

        def pair(kk2, _):
            for b in (0, 1):
                kk = 2 * kk2 + b
                issueL(jnp.minimum(kk + 1, nch - 1), 1 - b)
                waitL(b)

                @pl.when(kk2 >= 1)
                def _():
                    waitS(b)

                gth = pltpu.async_copy(xp_h.at[sb[b]], xrows, semG)

                def sg(r, _):
                    v = eb[b][pl.ds(r * 16, 16)]
                    sig = 1.0 / (1.0 + jnp.exp(-v))
                    rows_i = 4 * r + iot // 4
                    cols_i = 4 + iot % 4
                    plsc.store_scatter(mb[b], [rows_i, cols_i], sig)
                    plsc.store_scatter(ewb, [rows_i], sig, mask=(iot % 4) == 0)
                    return 0

                lax.fori_loop(0, (_KA * 4) // 16, sg, 0, unroll=8)
                gth.wait()

                def pe(e, _):
                    ei = jnp.full((16,), e, jnp.int32)
                    w = plsc.load_gather(ewb, [ei])
                    prod = xrows[e, :] * w
                    plsc.store_scatter(mb[b], [ei, iot], prod, mask=iot < 4)
                    return 0

                lax.fori_loop(0, _KA, pe, 0, unroll=8)
                pltpu.async_copy(mb[b], accA.at[db[b]], sS[b], add=True)
            return 0

        lax.fori_loop(0, nch // 2, pair, 0)
        waitS(0)
        waitS(1)
        waitL(0)
        plsc.subcore_barrier()

        for cc in range(_NC):
            @pl.when(c == cc)
            def _():
                _readout(accA, so, cc, s)

    f = pl.kernel(
        body,
        out_type=jax.ShapeDtypeStruct((_NC, _NPAD, 16), jnp.float32),
        mesh=_mesh(),
        compiler_params=pltpu.CompilerParams(
            needs_layout_passes=False, use_tc_tiling_on_sc=False),
        scratch_types=[
            pltpu.VMEM((_KA,), jnp.int32),
            pltpu.VMEM((_KA,), jnp.int32),
            pltpu.VMEM((_KA,), jnp.int32),
            pltpu.VMEM((_KA,), jnp.int32),
            pltpu.VMEM((_KA * 4,), jnp.float32),
            pltpu.VMEM((_KA * 4,), jnp.float32),
            pltpu.VMEM((_KA,), jnp.float32),
            pltpu.VMEM((_KA, 16), jnp.float32),
            pltpu.VMEM((_KA, 16), jnp.float32),
            pltpu.VMEM((_KA, 16), jnp.float32),
            pltpu.VMEM_SHARED((_NPAD, 16), jnp.float32),
            pltpu.VMEM((_ZR, 16), jnp.float32),
            pltpu.SemaphoreType.DMA,
            pltpu.SemaphoreType.DMA,
            pltpu.SemaphoreType.DMA,
            pltpu.SemaphoreType.DMA,
            pltpu.SemaphoreType.DMA,
        ],
    )
    return f(xp, esf, srcp, dstp)


def _pass_b(col, hA, hB, esf, srcp, dstp):
    def body(hA_h, hB_h, esf_h, src_h, dst_h, so,
             sbuf0, sbuf1, dbuf0, dbuf1, esb0, esb1, ewb,
             rows0, rows1, accB, zbuf,
             semL0, semL1, semG0, semG1, semS0, semS1):
        c = lax.axis_index("c")
        s = lax.axis_index("s")
        _zero_zbuf(zbuf, 32)
        _zero_acc(zbuf, accB, s)
        plsc.subcore_barrier()

        sb = (sbuf0, sbuf1)
        db = (dbuf0, dbuf1)
        eb = (esb0, esb1)
        rw = (rows0, rows1)
        sL = (semL0, semL1)
        sG = (semG0, semG1)
        sS = (semS0, semS1)

        ept = _EPAD // _NS
        nch = ept // _K
        iot = lax.iota(jnp.int32, 16)

        def issueL(kk, b):
            e0 = s * ept + kk * _K
            pltpu.async_copy(src_h.at[pl.ds(e0, _K)], sb[b], sL[b])
            pltpu.async_copy(dst_h.at[pl.ds(e0, _K)], db[b], sL[b])
            pltpu.async_copy(esf_h.at[pl.ds(e0 * 4, _K * 4)], eb[b], sL[b])

        def waitL(b):
            pltpu.make_async_copy(src_h.at[pl.ds(0, _K)], sb[b], sL[b]).wait()
            pltpu.make_async_copy(src_h.at[pl.ds(0, _K)], db[b], sL[b]).wait()
            pltpu.make_async_copy(esf_h.at[pl.ds(0, _K * 4)], eb[b], sL[b]).wait()

        def half(h_h):
            def waitS(b):
                pltpu.make_async_copy(h_h.at[pl.ds(0, _K)], rw[b], sS[b]).wait()

            issueL(0, 0)

            def pair(kk2, _):
                for b in (0, 1):
                    kk = 2 * kk2 + b
                    issueL(jnp.minimum(kk + 1, nch - 1), 1 - b)
                    waitL(b)

                    @pl.when(kk2 >= 1)
                    def _():
                        waitS(b)

                    gth = pltpu.async_copy(h_h.at[sb[b]], rw[b], sG[b])

                    for r in range(_K // 16):
                        idx = 64 * r + 4 * iot + col
                        v = plsc.load_gather(eb[b], [idx])
                        ewb[pl.ds(r * 16, 16)] = 1.0 / (1.0 + jnp.exp(-v))

                    gth.wait()

                    def pe(e, _):
                        w = plsc.load_gather(
                            ewb, [jnp.full((16,), e, jnp.int32)])
                        r0 = rw[b][e, pl.ds(0, 16)]
                        rw[b][e, pl.ds(0, 16)] = r0 * w
                        r1 = rw[b][e, pl.ds(16, 16)]
                        rw[b][e, pl.ds(16, 16)] = r1 * w
                        return 0

                    lax.fori_loop(0, _K, pe, 0, unroll=8)
                    pltpu.async_copy(rw[b], accB.at[db[b]], sS[b], add=True)
                return 0

            lax.fori_loop(0, nch // 2, pair, 0)
            waitS(0)
            waitS(1)
            waitL(0)

        @pl.when(c == 0)
        def _():
            half(hA_h)

        @pl.when(c == 1)
        def _():
            half(hB_h)

        plsc.subcore_barrier()
        for cc in range(_NC):
            @pl.when(c == cc)
            def _():
                _readout(accB, so, cc, s)

    f = pl.kernel(
        body,
        out_type=jax.ShapeDtypeStruct((_NC, _NPAD, 32), jnp.float32),
        mesh=_mesh(),
        compiler_params=pltpu.CompilerParams(
            needs_layout_passes=False, use_tc_tiling_on_sc=False),
        scratch_types=[
            pltpu.VMEM((_K,), jnp.int32),
            pltpu.VMEM((_K,), jnp.int32),
            pltpu.VMEM((_K,), jnp.int32),
            pltpu.VMEM((_K,), jnp.int32),
            pltpu.VMEM((_K * 4,), jnp.float32),
            pltpu.VMEM((_K * 4,), jnp.float32),
            pltpu.VMEM((_K,), jnp.float32),
            pltpu.VMEM((_K, 32), jnp.float32),
            pltpu.VMEM((_K, 32), jnp.float32),
            pltpu.VMEM_SHARED((_NPAD, 32), jnp.float32),
            pltpu.VMEM((_ZR, 32), jnp.float32),
            pltpu.SemaphoreType.DMA,
            pltpu.SemaphoreType.DMA,
            pltpu.SemaphoreType.DMA,
            pltpu.SemaphoreType.DMA,
            pltpu.SemaphoreType.DMA,
            pltpu.SemaphoreType.DMA,
        ],
    )
    return f(hA, hB, esf, srcp, dstp)


def _pass_c(hA, hB, batchp):
    def body(hA_h, hB_h, b_h, pool, cnto, hbuf, bbuf, ones, accP, accC, sem):
        c = lax.axis_index("c")
        s = lax.axis_index("s")

        def zh(k, _):
            hbuf[k, pl.ds(0, 16)] = jnp.zeros((16,), jnp.float32)
            hbuf[k, pl.ds(16, 16)] = jnp.zeros((16,), jnp.float32)
            ones[k, :] = jnp.zeros((16,), jnp.float32)
            return 0

        lax.fori_loop(0, _GPAD, zh, 0)

        @pl.when(s == 0)
        def _():
            pltpu.sync_copy(hbuf.at[pl.ds(0, _GPAD)], accP)
            pltpu.sync_copy(ones.at[pl.ds(0, _GPAD)], accC)

        plsc.subcore_barrier()

        def fo(k, _):
            ones[k, :] = jnp.ones((16,), jnp.float32)
            return 0

        lax.fori_loop(0, _CK, fo, 0)

        def half(cc, h_h):
            def chunk(kk, _):
                r0 = s * _RPT + kk * _CK
                pltpu.sync_copy(h_h.at[pl.ds(r0, _CK)], hbuf)
                pltpu.sync_copy(b_h.at[pl.ds(r0, _CK)], bbuf)
                pltpu.async_copy(hbuf, accP.at[bbuf], sem, add=True).wait()
                if cc == 0:
                    pltpu.async_copy(ones, accC.at[bbuf], sem, add=True).wait()
                return 0

            lax.fori_loop(0, _RPT // _CK, chunk, 0)

        @pl.when(c == 0)
        def _():
            half(0, hA_h)

        @pl.when(c == 1)
        def _():
            half(1, hB_h)

        plsc.subcore_barrier()
        for cc in range(_NC):
            @pl.when((c == cc) & (s == 0))
            def _():
                pltpu.sync_copy(accP, pool.at[cc])

        @pl.when((c == 0) & (s == 0))
        def _():
            pltpu.sync_copy(accC, cnto)

    f = pl.kernel(
        body,
        out_type=[
            jax.ShapeDtypeStruct((_NC, _GPAD, 32), jnp.float32),
            jax.ShapeDtypeStruct((_GPAD, 16), jnp.float32),
        ],
        mesh=_mesh(),
        compiler_params=pltpu.CompilerParams(
            needs_layout_passes=False, use_tc_tiling_on_sc=False),
        scratch_types=[
            pltpu.VMEM((_CK, 32), jnp.float32),
            pltpu.VMEM((_CK,), jnp.int32),
            pltpu.VMEM((_CK, 16), jnp.float32),
            pltpu.VMEM_SHARED((_GPAD, 32), jnp.float32),
            pltpu.VMEM_SHARED((_GPAD, 16), jnp.float32),
            pltpu.SemaphoreType.DMA,
        ],
    )
    return f(hA, hB, batchp)


_RT = 1568


def _tc1(xp, s1d, w1, b1, w2, w3, b3):
    def body(x_r, s_r, w1r, b1r, w2r, w3r, b3r, oa, ob, od):
        C = s_r[0] + s_r[1]
        S = C[:, 0:4]
        d0 = C[:, 4:5]
        xb = x_r[:, 0:4]
        h = (jnp.dot(S, w1r[...].T, preferred_element_type=jnp.float32)
             + d0 * b1r[...][None, :]
             - d0 * jnp.dot(xb, w2r[...].T, preferred_element_type=jnp.float32)
             + jnp.dot(xb, w3r[...].T, preferred_element_type=jnp.float32)
             + b3r[...][None, :])
        h = jnp.maximum(h, 0.0)
        oa[...] = h[:, 0:32]
        ob[...] = h[:, 32:64]
        od[...] = C

    return pl.pallas_call(
        body,
        grid=(_NPAD // _RT,),
        in_specs=[
            pl.BlockSpec((_RT, 16), lambda i: (i, 0)),
            pl.BlockSpec((_NC, _RT, 16), lambda i: (0, i, 0)),
            pl.BlockSpec((_CH, 4), lambda i: (0, 0)),
            pl.BlockSpec((_CH,), lambda i: (0,)),
            pl.BlockSpec((_CH, 4), lambda i: (0, 0)),
            pl.BlockSpec((_CH, 4), lambda i: (0, 0)),
            pl.BlockSpec((_CH,), lambda i: (0,)),
        ],
        out_specs=[
            pl.BlockSpec((_RT, 32), lambda i: (i, 0)),
            pl.BlockSpec((_RT, 32), lambda i: (i, 0)),
            pl.BlockSpec((_RT, 16), lambda i: (i, 0)),
        ],
        out_shape=[
            jax.ShapeDtypeStruct((_NPAD, 32), jnp.float32),
            jax.ShapeDtypeStruct((_NPAD, 32), jnp.float32),
            jax.ShapeDtypeStruct((_NPAD, 16), jnp.float32),
        ],
    )(xp, s1d, w1, b1, w2, w3, b3)


def _tc_mid(col, hA, hB, sp, dsum, w1, b1, w2, w3, b3):
    def body(ha_r, hb_r, s_r, d_r, w1r, b1r, w2r, w3r, b3r, oa, ob):
        h = jnp.concatenate([ha_r[...], hb_r[...]], axis=1)
        S = jnp.concatenate([s_r[0], s_r[1]], axis=1)
        d = d_r[:, 4 + col:5 + col]
        hn = (jnp.dot(S, w1r[...].T, preferred_element_type=jnp.float32)
              + d * b1r[...][None, :]
              - d * jnp.dot(h, w2r[...].T, preferred_element_type=jnp.float32)
              + jnp.dot(h, w3r[...].T, preferred_element_type=jnp.float32)
              + b3r[...][None, :])
        hn = jnp.maximum(hn, 0.0)
        oa[...] = hn[:, 0:32]
        ob[...] = hn[:, 32:64]

    return pl.pallas_call(
        body,
        grid=(_NPAD // _RT,),
        in_specs=[
            pl.BlockSpec((_RT, 32), lambda i: (i, 0)),
            pl.BlockSpec((_RT, 32), lambda i: (i, 0)),
            pl.BlockSpec((_NC, _RT, 32), lambda i: (0, i, 0)),
            pl.BlockSpec((_RT, 16), lambda i: (i, 0)),
            pl.BlockSpec((_CH, _CH), lambda i: (0, 0)),
            pl.BlockSpec((_CH,), lambda i: (0,)),
            pl.BlockSpec((_CH, _CH), lambda i: (0, 0)),
            pl.BlockSpec((_CH, _CH), lambda i: (0, 0)),
            pl.BlockSpec((_CH,), lambda i: (0,)),
        ],
        out_specs=[
            pl.BlockSpec((_RT, 32), lambda i: (i, 0)),
            pl.BlockSpec((_RT, 32), lambda i: (i, 0)),
        ],
        out_shape=[
            jax.ShapeDtypeStruct((_NPAD, 32), jnp.float32),
            jax.ShapeDtypeStruct((_NPAD, 32), jnp.float32),
        ],
    )(hA, hB, sp, dsum, w1, b1, w2, w3, b3)


def _tc_final(pool, cnt, w1, b1, w2, b2):
    def body(p_r, c_r, w1r, b1r, w2r, b2r, ogf, opred):
        sums = jnp.concatenate([p_r[0, 0:_G, :], p_r[1, 0:_G, :]], axis=1)
        cv = c_r[0:_G, 0:1]
        gf = sums / jnp.maximum(cv, 1.0)
        hid = jnp.maximum(
            jnp.dot(gf, w1r[...].T, preferred_element_type=jnp.float32)
            + b1r[...][None, :], 0.0)
        pred = (jnp.dot(hid, w2r[...].T, preferred_element_type=jnp.float32)
                + b2r[...][None, :])
        ogf[...] = gf
        opred[...] = pred

    return pl.pallas_call(
        body,
        out_shape=[
            jax.ShapeDtypeStruct((_G, _CH), jnp.float32),
            jax.ShapeDtypeStruct((_G, 2), jnp.float32),
        ],
    )(pool, cnt, w1, b1, w2, b2)


def kernel(x, edge_index, edge_score, batch,
           conv1_w1, conv1_b1, conv1_w2, conv1_w3, conv1_b3,
           conv2_w1, conv2_b1, conv2_w2, conv2_w3, conv2_b3,
           conv3_w1, conv3_b1, conv3_w2, conv3_w3, conv3_b3,
           conv4_w1, conv4_b1, conv4_w2, conv4_w3, conv4_b3,
           mlp_w1, mlp_b1, mlp_w2, mlp_b2):
    src = edge_index[0]
    dst = edge_index[1]
    pad_e = _EPAD - _E
    srcp = jnp.concatenate([src, jnp.zeros((pad_e,), jnp.int32)])
    dstp = jnp.concatenate([dst, jnp.full((pad_e,), _NPAD - 1, jnp.int32)])
    esp = jnp.concatenate(
        [edge_score, jnp.zeros((pad_e, 4), jnp.float32)], axis=0)
    esf = esp.reshape(-1)
    xp = jnp.zeros((_NPAD, 16), jnp.float32).at[:_N, 0:4].set(x)
    batchp = jnp.concatenate([batch, jnp.full((_NPAD - _N,), _G, jnp.int32)])

    s1d = _pass_a(xp, esf, srcp, dstp)
    hA, hB, dsum = _tc1(xp, s1d, conv1_w1, conv1_b1, conv1_w2,
                        conv1_w3, conv1_b3)
    sp = _pass_b(1, hA, hB, esf, srcp, dstp)
    hA, hB = _tc_mid(1, hA, hB, sp, dsum, conv2_w1, conv2_b1, conv2_w2,
                     conv2_w3, conv2_b3)
    sp = _pass_b(2, hA, hB, esf, srcp, dstp)
    hA, hB = _tc_mid(2, hA, hB, sp, dsum, conv3_w1, conv3_b1, conv3_w2,
                     conv3_w3, conv3_b3)
    sp = _pass_b(3, hA, hB, esf, srcp, dstp)
    hA, hB = _tc_mid(3, hA, hB, sp, dsum, conv4_w1, conv4_b1, conv4_w2,
                     conv4_w3, conv4_b3)
    pool, cnt = _pass_c(hA, hB, batchp)
    gf, pred = _tc_final(pool, cnt, mlp_w1, mlp_b1, mlp_w2, mlp_b2)
    return (gf, pred)


# speedup vs baseline: 5.9916x; 1.3789x over previous
import jax
import jax.numpy as jnp
from jax import lax
from jax.experimental import pallas as pl
from jax.experimental.pallas import tpu as pltpu
from jax.experimental.pallas import tpu_sc as plsc

_N = 50000
_E = 800000
_CH = 64
_G = 128
_NC = 2
_NS = 16
_NPAD = 50176
_EPAD = 802816
_K = 64
_ZR = 784
_RPT = _NPAD // _NS
_GPAD = 136
_CK = 448

_mesh = lambda: plsc.VectorSubcoreMesh(core_axis_name="c", subcore_axis_name="s")


def _zero_zbuf(zbuf, width):
    nv = width // 16

    def zb(k, _):
        for j in range(nv):
            zbuf[k, pl.ds(16 * j, 16)] = jnp.zeros((16,), jnp.float32)
        return 0

    lax.fori_loop(0, _ZR, zb, 0)


def _zero_acc(zbuf, acc, s):
    def cz(k, _):
        pltpu.sync_copy(zbuf, acc.at[pl.ds(s * _RPT + k * _ZR, _ZR)])
        return 0

    lax.fori_loop(0, _RPT // _ZR, cz, 0)


def _readout(acc, out, cc, s):
    def ro(k, _):
        r0 = s * _RPT + k * _ZR
        pltpu.sync_copy(acc.at[pl.ds(r0, _ZR)], out.at[cc, pl.ds(r0, _ZR)])
        return 0

    lax.fori_loop(0, _RPT // _ZR, ro, 0)


_KA = 448


def _tc_sig(esTp):
    RE = 16384

    def body(x_r, o0, o1, o2, o3):
        sig = 1.0 / (1.0 + jnp.exp(-x_r[...]))
        o0[...] = sig[0]
        o1[...] = sig[1]
        o2[...] = sig[2]
        o3[...] = sig[3]

    outs = pl.pallas_call(
        body,
        grid=(_EPAD // RE,),
        in_specs=[pl.BlockSpec((4, RE), lambda i: (0, i))],
        out_specs=[pl.BlockSpec((RE,), lambda i: (i,))] * 4,
        out_shape=[jax.ShapeDtypeStruct((_EPAD,), jnp.float32)] * 4,
    )(esTp)
    return outs


def _pass_a(xp, ew0, ew1, ew2, ew3, srcp, dstp):
    def body(xp_h, ew0_h, ew1_h, ew2_h, ew3_h, src_h, dst_h, so,
             sbuf0, sbuf1, dbuf0, dbuf1,
             eb00, eb01, eb02, eb03, eb10, eb11, eb12, eb13, xrows,
             msgb0, msgb1, accA, zbuf,
             semL0, semL1, semG, semS0, semS1):
        c = lax.axis_index("c")
        s = lax.axis_index("s")
        _zero_zbuf(zbuf, 16)
        _zero_acc(zbuf, accA, s)

        sb = (sbuf0, sbuf1)
        db = (dbuf0, dbuf1)
        eb = ((eb00, eb01, eb02, eb03), (eb10, eb11, eb12, eb13))
        ew_h = (ew0_h, ew1_h, ew2_h, ew3_h)
        mb = (msgb0, msgb1)
        sL = (semL0, semL1)
        sS = (semS0, semS1)

        def zs(e, _):
            msgb0[e, :] = jnp.zeros((16,), jnp.float32)
            msgb1[e, :] = jnp.zeros((16,), jnp.float32)
            return 0

        lax.fori_loop(0, _KA, zs, 0)
        plsc.subcore_barrier()

        ept = _EPAD // (_NC * _NS)
        tile_base = (c * _NS + s) * ept
        nch = ept // _KA
        iot = lax.iota(jnp.int32, 16)

        def issueL(kk, b):
            e0 = tile_base + kk * _KA
            pltpu.async_copy(src_h.at[pl.ds(e0, _KA)], sb[b], sL[b])
            pltpu.async_copy(dst_h.at[pl.ds(e0, _KA)], db[b], sL[b])
            for j in range(4):
                pltpu.async_copy(ew_h[j].at[pl.ds(e0, _KA)], eb[b][j], sL[b])

        def waitL(b):
            pltpu.make_async_copy(src_h.at[pl.ds(0, _KA)], sb[b], sL[b]).wait()
            pltpu.make_async_copy(src_h.at[pl.ds(0, _KA)], db[b], sL[b]).wait()
            for j in range(4):
                pltpu.make_async_copy(
                    ew0_h.at[pl.ds(0, _KA)], eb[b][j], sL[b]).wait()

        def waitS(b):
            pltpu.make_async_copy(xp_h.at[pl.ds(0, _KA)], mb[b], sS[b]).wait()

        issueL(0, 0)

        def pair(kk2, _):
            for b in (0, 1):
                kk = 2 * kk2 + b
                issueL(jnp.minimum(kk + 1, nch - 1), 1 - b)
                waitL(b)

                @pl.when(kk2 >= 1)
                def _():
                    waitS(b)

                gth = pltpu.async_copy(xp_h.at[sb[b]], xrows, semG)

                for j in range(4):
                    cj = jnp.full((16,), 4 + j, jnp.int32)
                    for r in range(_KA // 16):
                        v = eb[b][j][pl.ds(r * 16, 16)]
                        plsc.store_scatter(mb[b], [r * 16 + iot, cj], v)

                gth.wait()

                def pe(e, _):
                    ei = jnp.full((16,), e, jnp.int32)
                    w = plsc.load_gather(eb[b][0], [ei])
                    prod = xrows[e, :] * w
                    plsc.store_scatter(mb[b], [ei, iot], prod, mask=iot < 4)
                    return 0

                lax.fori_loop(0, _KA, pe, 0, unroll=8)
                pltpu.async_copy(mb[b], accA.at[db[b]], sS[b], add=True)
            return 0

        lax.fori_loop(0, nch // 2, pair, 0)
        waitS(0)
        waitS(1)
        waitL(0)
        plsc.subcore_barrier()

        for cc in range(_NC):
            @pl.when(c == cc)
            def _():
                _readout(accA, so, cc, s)

    f = pl.kernel(
        body,
        out_type=jax.ShapeDtypeStruct((_NC, _NPAD, 16), jnp.float32),
        mesh=_mesh(),
        compiler_params=pltpu.CompilerParams(
            needs_layout_passes=False, use_tc_tiling_on_sc=False),
        scratch_types=[
            pltpu.VMEM((_KA,), jnp.int32),
            pltpu.VMEM((_KA,), jnp.int32),
            pltpu.VMEM((_KA,), jnp.int32),
            pltpu.VMEM((_KA,), jnp.int32),
        ] + [pltpu.VMEM((_KA,), jnp.float32)] * 8 + [
            pltpu.VMEM((_KA, 16), jnp.float32),
            pltpu.VMEM((_KA, 16), jnp.float32),
            pltpu.VMEM((_KA, 16), jnp.float32),
            pltpu.VMEM_SHARED((_NPAD, 16), jnp.float32),
            pltpu.VMEM((_ZR, 16), jnp.float32),
            pltpu.SemaphoreType.DMA,
            pltpu.SemaphoreType.DMA,
            pltpu.SemaphoreType.DMA,
            pltpu.SemaphoreType.DMA,
            pltpu.SemaphoreType.DMA,
        ],
    )
    return f(xp, ew0, ew1, ew2, ew3, srcp, dstp)


def _pass_b(hA, hB, ewc, srcp, dstp):
    def body(hA_h, hB_h, ew_h, src_h, dst_h, so,
             sbuf0, sbuf1, dbuf0, dbuf1, esb0, esb1,
             rows0, rows1, accB, zbuf,
             semL0, semL1, semG0, semG1, semS0, semS1):
        c = lax.axis_index("c")
        s = lax.axis_index("s")
        _zero_zbuf(zbuf, 32)
        _zero_acc(zbuf, accB, s)
        plsc.subcore_barrier()

        sb = (sbuf0, sbuf1)
        db = (dbuf0, dbuf1)
        eb = (esb0, esb1)
        rw = (rows0, rows1)
        sL = (semL0, semL1)
        sG = (semG0, semG1)
        sS = (semS0, semS1)

        ept = _EPAD // _NS
        nch = ept // _K
        iot = lax.iota(jnp.int32, 16)

        def issueL(kk, b):
            e0 = s * ept + kk * _K
            pltpu.async_copy(src_h.at[pl.ds(e0, _K)], sb[b], sL[b])
            pltpu.async_copy(dst_h.at[pl.ds(e0, _K)], db[b], sL[b])
            pltpu.async_copy(ew_h.at[pl.ds(e0, _K)], eb[b], sL[b])

        def waitL(b):
            pltpu.make_async_copy(src_h.at[pl.ds(0, _K)], sb[b], sL[b]).wait()
            pltpu.make_async_copy(src_h.at[pl.ds(0, _K)], db[b], sL[b]).wait()
            pltpu.make_async_copy(ew_h.at[pl.ds(0, _K)], eb[b], sL[b]).wait()

        def half(h_h):
            def waitS(b):
                pltpu.make_async_copy(h_h.at[pl.ds(0, _K)], rw[b], sS[b]).wait()

            issueL(0, 0)

            def pair(kk2, _):
                for b in (0, 1):
                    kk = 2 * kk2 + b
                    issueL(jnp.minimum(kk + 1, nch - 1), 1 - b)
                    waitL(b)

                    @pl.when(kk2 >= 1)
                    def _():
                        waitS(b)

                    gth = pltpu.async_copy(h_h.at[sb[b]], rw[b], sG[b])
                    gth.wait()

                    def pe(e, _):
                        w = plsc.load_gather(
                            eb[b], [jnp.full((16,), e, jnp.int32)])
                        r0 = rw[b][e, pl.ds(0, 16)]
                        rw[b][e, pl.ds(0, 16)] = r0 * w
                        r1 = rw[b][e, pl.ds(16, 16)]
                        rw[b][e, pl.ds(16, 16)] = r1 * w
                        return 0

                    lax.fori_loop(0, _K, pe, 0, unroll=8)
                    pltpu.async_copy(rw[b], accB.at[db[b]], sS[b], add=True)
                return 0

            lax.fori_loop(0, nch // 2, pair, 0)
            waitS(0)
            waitS(1)
            waitL(0)

        @pl.when(c == 0)
        def _():
            half(hA_h)

        @pl.when(c == 1)
        def _():
            half(hB_h)

        plsc.subcore_barrier()
        for cc in range(_NC):
            @pl.when(c == cc)
            def _():
                _readout(accB, so, cc, s)

    f = pl.kernel(
        body,
        out_type=jax.ShapeDtypeStruct((_NC, _NPAD, 32), jnp.float32),
        mesh=_mesh(),
        compiler_params=pltpu.CompilerParams(
            needs_layout_passes=False, use_tc_tiling_on_sc=False),
        scratch_types=[
            pltpu.VMEM((_K,), jnp.int32),
            pltpu.VMEM((_K,), jnp.int32),
            pltpu.VMEM((_K,), jnp.int32),
            pltpu.VMEM((_K,), jnp.int32),
            pltpu.VMEM((_K,), jnp.float32),
            pltpu.VMEM((_K,), jnp.float32),
            pltpu.VMEM((_K, 32), jnp.float32),
            pltpu.VMEM((_K, 32), jnp.float32),
            pltpu.VMEM_SHARED((_NPAD, 32), jnp.float32),
            pltpu.VMEM((_ZR, 32), jnp.float32),
            pltpu.SemaphoreType.DMA,
            pltpu.SemaphoreType.DMA,
            pltpu.SemaphoreType.DMA,
            pltpu.SemaphoreType.DMA,
            pltpu.SemaphoreType.DMA,
            pltpu.SemaphoreType.DMA,
        ],
    )
    return f(hA, hB, ewc, srcp, dstp)


def _pass_c(hA, hB, batchp):
    def body(hA_h, hB_h, b_h, pool, cnto, hbuf, bbuf, ones, accP, accC, sem):
        c = lax.axis_index("c")
        s = lax.axis_index("s")

        def zh(k, _):
            hbuf[k, pl.ds(0, 16)] = jnp.zeros((16,), jnp.float32)
            hbuf[k, pl.ds(16, 16)] = jnp.zeros((16,), jnp.float32)
            ones[k, :] = jnp.zeros((16,), jnp.float32)
            return 0

        lax.fori_loop(0, _GPAD, zh, 0)

        @pl.when(s == 0)
        def _():
            pltpu.sync_copy(hbuf.at[pl.ds(0, _GPAD)], accP)
            pltpu.sync_copy(ones.at[pl.ds(0, _GPAD)], accC)

        plsc.subcore_barrier()

        def fo(k, _):
            ones[k, :] = jnp.ones((16,), jnp.float32)
            return 0

        lax.fori_loop(0, _CK, fo, 0)

        def half(cc, h_h):
            def chunk(kk, _):
                r0 = s * _RPT + kk * _CK
                pltpu.sync_copy(h_h.at[pl.ds(r0, _CK)], hbuf)
                pltpu.sync_copy(b_h.at[pl.ds(r0, _CK)], bbuf)
                pltpu.async_copy(hbuf, accP.at[bbuf], sem, add=True).wait()
                if cc == 0:
                    pltpu.async_copy(ones, accC.at[bbuf], sem, add=True).wait()
                return 0

            lax.fori_loop(0, _RPT // _CK, chunk, 0)

        @pl.when(c == 0)
        def _():
            half(0, hA_h)

        @pl.when(c == 1)
        def _():
            half(1, hB_h)

        plsc.subcore_barrier()
        for cc in range(_NC):
            @pl.when((c == cc) & (s == 0))
            def _():
                pltpu.sync_copy(accP, pool.at[cc])

        @pl.when((c == 0) & (s == 0))
        def _():
            pltpu.sync_copy(accC, cnto)

    f = pl.kernel(
        body,
        out_type=[
            jax.ShapeDtypeStruct((_NC, _GPAD, 32), jnp.float32),
            jax.ShapeDtypeStruct((_GPAD, 16), jnp.float32),
        ],
        mesh=_mesh(),
        compiler_params=pltpu.CompilerParams(
            needs_layout_passes=False, use_tc_tiling_on_sc=False),
        scratch_types=[
            pltpu.VMEM((_CK, 32), jnp.float32),
            pltpu.VMEM((_CK,), jnp.int32),
            pltpu.VMEM((_CK, 16), jnp.float32),
            pltpu.VMEM_SHARED((_GPAD, 32), jnp.float32),
            pltpu.VMEM_SHARED((_GPAD, 16), jnp.float32),
            pltpu.SemaphoreType.DMA,
        ],
    )
    return f(hA, hB, batchp)


_RT = 1568


def _tc1(xp, s1d, w1, b1, w2, w3, b3):
    def body(x_r, s_r, w1r, b1r, w2r, w3r, b3r, oa, ob, od):
        C = s_r[0] + s_r[1]
        S = C[:, 0:4]
        d0 = C[:, 4:5]
        xb = x_r[:, 0:4]
        h = (jnp.dot(S, w1r[...].T, preferred_element_type=jnp.float32)
             + d0 * b1r[...][None, :]
             - d0 * jnp.dot(xb, w2r[...].T, preferred_element_type=jnp.float32)
             + jnp.dot(xb, w3r[...].T, preferred_element_type=jnp.float32)
             + b3r[...][None, :])
        h = jnp.maximum(h, 0.0)
        oa[...] = h[:, 0:32]
        ob[...] = h[:, 32:64]
        od[...] = C

    return pl.pallas_call(
        body,
        grid=(_NPAD // _RT,),
        in_specs=[
            pl.BlockSpec((_RT, 16), lambda i: (i, 0)),
            pl.BlockSpec((_NC, _RT, 16), lambda i: (0, i, 0)),
            pl.BlockSpec((_CH, 4), lambda i: (0, 0)),
            pl.BlockSpec((_CH,), lambda i: (0,)),
            pl.BlockSpec((_CH, 4), lambda i: (0, 0)),
            pl.BlockSpec((_CH, 4), lambda i: (0, 0)),
            pl.BlockSpec((_CH,), lambda i: (0,)),
        ],
        out_specs=[
            pl.BlockSpec((_RT, 32), lambda i: (i, 0)),
            pl.BlockSpec((_RT, 32), lambda i: (i, 0)),
            pl.BlockSpec((_RT, 16), lambda i: (i, 0)),
        ],
        out_shape=[
            jax.ShapeDtypeStruct((_NPAD, 32), jnp.float32),
            jax.ShapeDtypeStruct((_NPAD, 32), jnp.float32),
            jax.ShapeDtypeStruct((_NPAD, 16), jnp.float32),
        ],
    )(xp, s1d, w1, b1, w2, w3, b3)


def _tc_mid(col, hA, hB, sp, dsum, w1, b1, w2, w3, b3):
    def body(ha_r, hb_r, s_r, d_r, w1r, b1r, w2r, w3r, b3r, oa, ob):
        h = jnp.concatenate([ha_r[...], hb_r[...]], axis=1)
        S = jnp.concatenate([s_r[0], s_r[1]], axis=1)
        d = d_r[:, 4 + col:5 + col]
        hn = (jnp.dot(S, w1r[...].T, preferred_element_type=jnp.float32)
              + d * b1r[...][None, :]
              - d * jnp.dot(h, w2r[...].T, preferred_element_type=jnp.float32)
              + jnp.dot(h, w3r[...].T, preferred_element_type=jnp.float32)
              + b3r[...][None, :])
        hn = jnp.maximum(hn, 0.0)
        oa[...] = hn[:, 0:32]
        ob[...] = hn[:, 32:64]

    return pl.pallas_call(
        body,
        grid=(_NPAD // _RT,),
        in_specs=[
            pl.BlockSpec((_RT, 32), lambda i: (i, 0)),
            pl.BlockSpec((_RT, 32), lambda i: (i, 0)),
            pl.BlockSpec((_NC, _RT, 32), lambda i: (0, i, 0)),
            pl.BlockSpec((_RT, 16), lambda i: (i, 0)),
            pl.BlockSpec((_CH, _CH), lambda i: (0, 0)),
            pl.BlockSpec((_CH,), lambda i: (0,)),
            pl.BlockSpec((_CH, _CH), lambda i: (0, 0)),
            pl.BlockSpec((_CH, _CH), lambda i: (0, 0)),
            pl.BlockSpec((_CH,), lambda i: (0,)),
        ],
        out_specs=[
            pl.BlockSpec((_RT, 32), lambda i: (i, 0)),
            pl.BlockSpec((_RT, 32), lambda i: (i, 0)),
        ],
        out_shape=[
            jax.ShapeDtypeStruct((_NPAD, 32), jnp.float32),
            jax.ShapeDtypeStruct((_NPAD, 32), jnp.float32),
        ],
    )(hA, hB, sp, dsum, w1, b1, w2, w3, b3)


def _tc_final(pool, cnt, w1, b1, w2, b2):
    def body(p_r, c_r, w1r, b1r, w2r, b2r, ogf, opred):
        sums = jnp.concatenate([p_r[0, 0:_G, :], p_r[1, 0:_G, :]], axis=1)
        cv = c_r[0:_G, 0:1]
        gf = sums / jnp.maximum(cv, 1.0)
        hid = jnp.maximum(
            jnp.dot(gf, w1r[...].T, preferred_element_type=jnp.float32)
            + b1r[...][None, :], 0.0)
        pred = (jnp.dot(hid, w2r[...].T, preferred_element_type=jnp.float32)
                + b2r[...][None, :])
        ogf[...] = gf
        opred[...] = pred

    return pl.pallas_call(
        body,
        out_shape=[
            jax.ShapeDtypeStruct((_G, _CH), jnp.float32),
            jax.ShapeDtypeStruct((_G, 2), jnp.float32),
        ],
    )(pool, cnt, w1, b1, w2, b2)


def kernel(x, edge_index, edge_score, batch,
           conv1_w1, conv1_b1, conv1_w2, conv1_w3, conv1_b3,
           conv2_w1, conv2_b1, conv2_w2, conv2_w3, conv2_b3,
           conv3_w1, conv3_b1, conv3_w2, conv3_w3, conv3_b3,
           conv4_w1, conv4_b1, conv4_w2, conv4_w3, conv4_b3,
           mlp_w1, mlp_b1, mlp_w2, mlp_b2):
    src = edge_index[0]
    dst = edge_index[1]
    pad_e = _EPAD - _E
    srcp = jnp.concatenate([src, jnp.zeros((pad_e,), jnp.int32)])
    dstp = jnp.concatenate([dst, jnp.full((pad_e,), _NPAD - 1, jnp.int32)])
    esTp = jnp.pad(edge_score.T, ((0, 0), (0, pad_e)))
    ew0, ew1, ew2, ew3 = _tc_sig(esTp)
    xp = jnp.zeros((_NPAD, 16), jnp.float32).at[:_N, 0:4].set(x)
    batchp = jnp.concatenate([batch, jnp.full((_NPAD - _N,), _G, jnp.int32)])

    s1d = _pass_a(xp, ew0, ew1, ew2, ew3, srcp, dstp)
    hA, hB, dsum = _tc1(xp, s1d, conv1_w1, conv1_b1, conv1_w2,
                        conv1_w3, conv1_b3)
    sp = _pass_b(hA, hB, ew1, srcp, dstp)
    hA, hB = _tc_mid(1, hA, hB, sp, dsum, conv2_w1, conv2_b1, conv2_w2,
                     conv2_w3, conv2_b3)
    sp = _pass_b(hA, hB, ew2, srcp, dstp)
    hA, hB = _tc_mid(2, hA, hB, sp, dsum, conv3_w1, conv3_b1, conv3_w2,
                     conv3_w3, conv3_b3)
    sp = _pass_b(hA, hB, ew3, srcp, dstp)
    hA, hB = _tc_mid(3, hA, hB, sp, dsum, conv4_w1, conv4_b1, conv4_w2,
                     conv4_w3, conv4_b3)
    pool, cnt = _pass_c(hA, hB, batchp)
    gf, pred = _tc_final(pool, cnt, mlp_w1, mlp_b1, mlp_w2, mlp_b2)
    return (gf, pred)
